# baseline (speedup 1.0000x reference)
import functools

import jax
import jax.numpy as jnp
from jax import lax
from jax.experimental import pallas as pl
from jax.experimental.pallas import tpu as pltpu
from jax.experimental.pallas import tpu_sc as plsc

N_NODES = 100000
N_EDGES = 3200000
SD = 16
VD = 8
FW = 32
MW = 24
AW = 32

NC = 2
NS = 16
NW = NC * NS

EPW = N_EDGES // NW
G1_CHUNK = 400
G1_BATCH = 80
G1_NB = G1_CHUNK // G1_BATCH
G1_SUPER = 10
G1_SUP_E = G1_CHUNK * G1_SUPER
G1_ITERS = EPW // G1_SUP_E
G1_UNROLL = 5

NPS = N_NODES // NC
NP_PAD = 50048
DUMMY_ROW = 50040
RPT = NP_PAD // NS
EPT = N_EDGES // NS
S3_CHUNK = 400
S3_BATCH = 80
S3_NB = S3_CHUNK // S3_BATCH
S3_RC = S3_CHUNK // 16
S3_ITERS = EPT // S3_CHUNK

MLP_ROWS = 2000
MLP_GRID = (N_EDGES // 4) // MLP_ROWS

_SC_PARAMS = pltpu.CompilerParams(use_tc_tiling_on_sc=False,
                                  needs_layout_passes=False)


def _prep_body(hs_ref, hv_ref, pos_ref, ori_ref, f_ref):
    a2 = 2.0 * ori_ref[...]
    rows = hs_ref.shape[0]
    f_ref[...] = jnp.concatenate(
        [hs_ref[...], hv_ref[...], pos_ref[...],
         jnp.cos(a2), jnp.sin(a2),
         jnp.zeros((rows, 4), jnp.float32)], axis=1)


def _build_node_table(h_scalar, h_vector, pos, orientation):
    blk = 2000
    grid = N_NODES // blk
    return pl.pallas_call(
        _prep_body,
        grid=(grid,),
        in_specs=[
            pl.BlockSpec((blk, SD), lambda i: (i, 0)),
            pl.BlockSpec((blk, VD), lambda i: (i, 0)),
            pl.BlockSpec((blk, 2), lambda i: (i, 0)),
            pl.BlockSpec((blk, 1), lambda i: (i, 0)),
        ],
        out_specs=pl.BlockSpec((blk, FW), lambda i: (i, 0)),
        out_shape=jax.ShapeDtypeStruct((N_NODES, FW), jnp.float32),
    )(h_scalar, h_vector, pos, orientation)


def _geo_group(rows_s, rows_d, k):
    r = k * 16 + lax.iota(jnp.int32, 16)

    def ld(ref, col):
        return plsc.load_gather(ref, [r, jnp.full((16,), col, jnp.int32)])

    def st(ref, col, val):
        plsc.store_scatter(ref, [r, jnp.full((16,), col, jnp.int32)], val)

    px_s, py_s, cb, sb = ld(rows_s, 24), ld(rows_s, 25), ld(rows_s, 26), ld(rows_s, 27)
    px_d, py_d, ca, sa = ld(rows_d, 24), ld(rows_d, 25), ld(rows_d, 26), ld(rows_d, 27)
    dx = px_s - px_d
    dy = py_s - py_d
    r2 = dx * dx + dy * dy
    nz = r2 > 0.0
    i = plsc.bitcast(r2, jnp.int32)
    i = jnp.int32(0x5F3759DF) - lax.shift_right_logical(i, 1)
    y = plsc.bitcast(i, jnp.float32)
    half = -0.5 * r2
    for _ in range(3):
        y = y * (1.5 + half * y * y)
    y = jnp.where(nz, y, 0.0)
    dist = r2 * y + 1e-6
    inv = y * y
    c2p = jnp.where(nz, (dx * dx - dy * dy) * inv, 1.0)
    s2p = jnp.where(nz, (2.0 * dx * dy) * inv, 0.0)
    cc = c2p * ca + s2p * sa
    ss = s2p * ca - c2p * sa
    cr = cb * ca + sb * sa
    sr = sb * ca - cb * sa
    st(rows_d, 16, dist)
    st(rows_d, 17, cc)
    st(rows_d, 18, ss)
    for c in range(VD):
        v = ld(rows_s, SD + c)
        st(rows_s, SD + c, v * cr)
        st(rows_s, SD + VD + c, v * sr)


def _gather_body(f_hbm, src_hbm, dst_hbm, gs_hbm, gd_hbm,
                 idx_s, idx_d, rows_s, rows_d, sem_g0, sem_g1, sem_w0,
                 sem_w1):
    sem_g = (sem_g0, sem_g1)
    sem_w = (sem_w0, sem_w1)
    wid = lax.axis_index("s") * NC + lax.axis_index("c")
    base = wid * EPW

    def fire_gathers(b, c):
        cps = []
        for j in range(G1_NB):
            isl = pl.ds(c * G1_CHUNK + j * G1_BATCH, G1_BATCH)
            osl = pl.ds(j * G1_BATCH, G1_BATCH)
            cps.append(pltpu.async_copy(
                f_hbm.at[idx_s.at[isl]], rows_s.at[b].at[osl], sem_g[b]))
            cps.append(pltpu.async_copy(
                f_hbm.at[idx_d.at[isl]], rows_d.at[b].at[osl], sem_g[b]))
        return cps

    def geo(b):
        def geo_body(k, carry2):
            for u in range(G1_UNROLL):
                _geo_group(rows_s.at[b], rows_d.at[b], k * G1_UNROLL + u)
            return carry2
        lax.fori_loop(0, G1_CHUNK // 16 // G1_UNROLL, geo_body, 0)

    def drain_writes(b):
        pltpu.make_async_copy(
            rows_s.at[b], gs_hbm.at[pl.ds(base, G1_CHUNK)], sem_w[b]).wait()
        pltpu.make_async_copy(
            rows_d.at[b], gd_hbm.at[pl.ds(base, G1_CHUNK)], sem_w[b]).wait()

    for b in range(2):
        pltpu.async_copy(rows_s.at[b], gs_hbm.at[pl.ds(base, G1_CHUNK)],
                         sem_w[b])
        pltpu.async_copy(rows_d.at[b], gd_hbm.at[pl.ds(base, G1_CHUNK)],
                         sem_w[b])

    def it_body(it, carry):
        soff = base + it * G1_SUP_E
        pltpu.sync_copy(src_hbm.at[pl.ds(soff, G1_SUP_E)], idx_s)
        pltpu.sync_copy(dst_hbm.at[pl.ds(soff, G1_SUP_E)], idx_d)
        drain_writes(0)
        cps = fire_gathers(0, 0)
        for c in range(G1_SUPER):
            b = c % 2
            for cp in cps:
                cp.wait()
            if c + 1 < G1_SUPER:
                drain_writes(1 - b)
                cps = fire_gathers(1 - b, c + 1)
            geo(b)
            off = soff + c * G1_CHUNK
            pltpu.async_copy(rows_s.at[b], gs_hbm.at[pl.ds(off, G1_CHUNK)],
                             sem_w[b])
            pltpu.async_copy(rows_d.at[b], gd_hbm.at[pl.ds(off, G1_CHUNK)],
                             sem_w[b])
        return carry

    lax.fori_loop(0, G1_ITERS, it_body, 0)
    drain_writes(0)
    drain_writes(1)


@functools.cache
def _gather_edges():
    return pl.kernel(
        _gather_body,
        out_type=(jax.ShapeDtypeStruct((N_EDGES, FW), jnp.float32),
                  jax.ShapeDtypeStruct((N_EDGES, FW), jnp.float32)),
        mesh=plsc.VectorSubcoreMesh(core_axis_name="c", subcore_axis_name="s",
                                    num_cores=NC, num_subcores=NS),
        scratch_types=[
            pltpu.VMEM((G1_SUP_E,), jnp.int32),
            pltpu.VMEM((G1_SUP_E,), jnp.int32),
            pltpu.VMEM((2, G1_CHUNK, FW), jnp.float32),
            pltpu.VMEM((2, G1_CHUNK, FW), jnp.float32),
            pltpu.SemaphoreType.DMA,
            pltpu.SemaphoreType.DMA,
            pltpu.SemaphoreType.DMA,
            pltpu.SemaphoreType.DMA,
        ],
        compiler_params=_SC_PARAMS,
    )


def _mlp_body(gs_ref, gd_ref, ws_ref, wd_ref, b1_ref, w2_ref, b2_ref,
              out_ref):
    dot = functools.partial(jnp.dot, preferred_element_type=jnp.float32)
    pre = (dot(gs_ref[...], ws_ref[...]) + dot(gd_ref[...], wd_ref[...])
           + b1_ref[...])
    h = pre * jax.nn.sigmoid(pre)
    out_ref[...] = dot(h, w2_ref[...]) + b2_ref[...]


def _run_mlp(gs, gd, ws, wd, b1, w2, b2):
    full = lambda shape: pl.BlockSpec(shape, lambda i: (0, 0))
    return pl.pallas_call(
        _mlp_body,
        grid=(MLP_GRID,),
        in_specs=[
            pl.BlockSpec((MLP_ROWS, 128), lambda i: (i, 0)),
            pl.BlockSpec((MLP_ROWS, 128), lambda i: (i, 0)),
            full((128, 128)), full((128, 128)), full((1, 128)),
            full((128, 128)), full((1, 128)),
        ],
        out_specs=pl.BlockSpec((MLP_ROWS, 128), lambda i: (i, 0)),
        out_shape=jax.ShapeDtypeStruct((N_EDGES // 4, 128), jnp.float32),
    )(gs, gd, ws, wd, b1, w2, b2)


def _scatter_body(msg_hbm, dst_hbm, init_hbm, out_hbm,
                  acc, msg_v, idx_lin, idx2, sem):
    c = lax.axis_index("c")
    s = lax.axis_index("s")
    nbase = c * NPS
    pltpu.sync_copy(init_hbm.at[c, pl.ds(s * RPT, RPT)],
                    acc.at[pl.ds(s * RPT, RPT)])
    plsc.subcore_barrier()
    msg_rows = msg_hbm

    def it_body(it, carry):
        ebase = s * EPT + it * S3_CHUNK
        pltpu.sync_copy(dst_hbm.at[pl.ds(ebase, S3_CHUNK)], idx_lin)
        pltpu.sync_copy(msg_rows.at[pl.ds(ebase, S3_CHUNK)], msg_v)
        for k in range(S3_RC):
            loc = idx_lin[pl.ds(k * 16, 16)] - nbase
            oob = (loc < 0) | (loc >= NPS)
            loc = jnp.where(oob, DUMMY_ROW, loc)
            o = k * 16
            idx2[o // S3_BATCH, pl.ds(o % S3_BATCH, 16)] = loc
        for b in range(S3_NB):
            pltpu.sync_copy(msg_v.at[pl.ds(b * S3_BATCH, S3_BATCH)],
                            acc.at[idx2.at[b]], add=True)
        return carry

    lax.fori_loop(0, S3_ITERS, it_body, 0)
    plsc.subcore_barrier()
    pltpu.sync_copy(acc.at[pl.ds(s * RPT, RPT)],
                    out_hbm.at[c, pl.ds(s * RPT, RPT)])


@functools.cache
def _scatter_edges():
    return pl.kernel(
        _scatter_body,
        out_type=jax.ShapeDtypeStruct((NC, NP_PAD, AW), jnp.float32),
        mesh=plsc.VectorSubcoreMesh(core_axis_name="c", subcore_axis_name="s",
                                    num_cores=NC, num_subcores=NS),
        scratch_types=[
            pltpu.VMEM_SHARED((NP_PAD, AW), jnp.float32),
            pltpu.VMEM((S3_CHUNK, AW), jnp.float32),
            pltpu.VMEM((S3_CHUNK,), jnp.int32),
            pltpu.VMEM((S3_NB, S3_BATCH), jnp.int32),
            pltpu.SemaphoreType.DMA,
        ],
        compiler_params=_SC_PARAMS,
    )


def kernel(h_scalar, h_vector, edge_index, pos, orientation, W1, b1, W2, b2):
    wa = W1[:SD]
    wb = W1[SD:2 * SD]
    wv = W1[2 * SD:2 * SD + VD]
    wq = jnp.stack([wv[1::2], -wv[0::2]], axis=1).reshape(VD, MW)
    wg = W1[2 * SD + VD:]
    ws32 = jnp.concatenate(
        [jnp.concatenate([wa, wv, wq], axis=0),
         jnp.zeros((FW, FW - MW), jnp.float32)], axis=1)
    wd32 = jnp.concatenate(
        [jnp.concatenate([wb, wg, jnp.zeros((FW - SD - 3, MW), jnp.float32)],
                         axis=0),
         jnp.zeros((FW, FW - MW), jnp.float32)], axis=1)
    w232 = jnp.concatenate(
        [jnp.concatenate([W2, jnp.zeros((FW - MW, MW), jnp.float32)], axis=0),
         jnp.zeros((FW, FW - MW), jnp.float32)], axis=1)
    eye4 = jnp.eye(4, dtype=jnp.float32)
    ws_big = jnp.kron(eye4, ws32)
    wd_big = jnp.kron(eye4, wd32)
    w2_big = jnp.kron(eye4, w232)
    b1p = jnp.concatenate([b1, jnp.zeros((FW - MW,), jnp.float32)])
    b2p = jnp.concatenate([b2, jnp.zeros((FW - MW,), jnp.float32)])
    b1_big = jnp.tile(b1p, 4).reshape(1, 128)
    b2_big = jnp.tile(b2p, 4).reshape(1, 128)

    src = edge_index[0]
    dst = edge_index[1]
    f_table = _build_node_table(h_scalar, h_vector, pos, orientation)
    gs, gd = _gather_edges()(f_table, src, dst)
    gs = gs.reshape(N_EDGES // 4, 128)
    gd = gd.reshape(N_EDGES // 4, 128)
    msg = _run_mlp(gs, gd, ws_big, wd_big, b1_big, w2_big, b2_big)
    msg = msg.reshape(N_EDGES, AW)

    base = jnp.concatenate(
        [h_scalar, h_vector, jnp.zeros((N_NODES, AW - MW), jnp.float32)],
        axis=1).reshape(NC, NPS, AW)
    init = jnp.concatenate(
        [base, jnp.zeros((NC, NP_PAD - NPS, AW), jnp.float32)], axis=1)
    acc = _scatter_edges()(msg, dst, init)

    out_scalar = jnp.concatenate([acc[0, :NPS, :SD], acc[1, :NPS, :SD]], axis=0)
    out_vector = jnp.concatenate(
        [acc[0, :NPS, SD:MW], acc[1, :NPS, SD:MW]], axis=0)
    return (out_scalar, out_vector)


# speedup vs baseline: 1.0019x; 1.0019x over previous
import functools

import jax
import jax.numpy as jnp
from jax import lax
from jax.experimental import pallas as pl
from jax.experimental.pallas import tpu as pltpu
from jax.experimental.pallas import tpu_sc as plsc

N_NODES = 100000
N_EDGES = 3200000
SD = 16
VD = 8
FW = 32
MW = 24
AW = 32

NC = 2
NS = 16
NW = NC * NS

EPW = N_EDGES // NW
G1_CHUNK = 400
G1_BATCH = 80
G1_NB = G1_CHUNK // G1_BATCH
G1_SUPER = 10
G1_SUP_E = G1_CHUNK * G1_SUPER
G1_ITERS = EPW // G1_SUP_E
G1_UNROLL = 5

NPS = N_NODES // NC
NP_PAD = 50048
DUMMY_ROW = 50040
RPT = NP_PAD // NS
EPT = N_EDGES // NS
S3_CHUNK = 400
S3_BATCH = 80
S3_NB = S3_CHUNK // S3_BATCH
S3_RC = S3_CHUNK // 16
S3_SUPER = 10
S3_SUP_E = S3_CHUNK * S3_SUPER
S3_ITERS = EPT // S3_SUP_E

MLP_ROWS = 2000
MLP_GRID = (N_EDGES // 4) // MLP_ROWS

_SC_PARAMS = pltpu.CompilerParams(use_tc_tiling_on_sc=False,
                                  needs_layout_passes=False)


def _prep_body(hs_ref, hv_ref, pos_ref, ori_ref, f_ref):
    a2 = 2.0 * ori_ref[...]
    rows = hs_ref.shape[0]
    f_ref[...] = jnp.concatenate(
        [hs_ref[...], hv_ref[...], pos_ref[...],
         jnp.cos(a2), jnp.sin(a2),
         jnp.zeros((rows, 4), jnp.float32)], axis=1)


def _build_node_table(h_scalar, h_vector, pos, orientation):
    blk = 2000
    grid = N_NODES // blk
    return pl.pallas_call(
        _prep_body,
        grid=(grid,),
        in_specs=[
            pl.BlockSpec((blk, SD), lambda i: (i, 0)),
            pl.BlockSpec((blk, VD), lambda i: (i, 0)),
            pl.BlockSpec((blk, 2), lambda i: (i, 0)),
            pl.BlockSpec((blk, 1), lambda i: (i, 0)),
        ],
        out_specs=pl.BlockSpec((blk, FW), lambda i: (i, 0)),
        out_shape=jax.ShapeDtypeStruct((N_NODES, FW), jnp.float32),
    )(h_scalar, h_vector, pos, orientation)


def _geo_group(rows_s, rows_d, k):
    r = k * 16 + lax.iota(jnp.int32, 16)

    def ld(ref, col):
        return plsc.load_gather(ref, [r, jnp.full((16,), col, jnp.int32)])

    def st(ref, col, val):
        plsc.store_scatter(ref, [r, jnp.full((16,), col, jnp.int32)], val)

    px_s, py_s, cb, sb = ld(rows_s, 24), ld(rows_s, 25), ld(rows_s, 26), ld(rows_s, 27)
    px_d, py_d, ca, sa = ld(rows_d, 24), ld(rows_d, 25), ld(rows_d, 26), ld(rows_d, 27)
    dx = px_s - px_d
    dy = py_s - py_d
    r2 = dx * dx + dy * dy
    nz = r2 > 0.0
    i = plsc.bitcast(r2, jnp.int32)
    i = jnp.int32(0x5F3759DF) - lax.shift_right_logical(i, 1)
    y = plsc.bitcast(i, jnp.float32)
    half = -0.5 * r2
    for _ in range(3):
        y = y * (1.5 + half * y * y)
    y = jnp.where(nz, y, 0.0)
    dist = r2 * y + 1e-6
    inv = y * y
    c2p = jnp.where(nz, (dx * dx - dy * dy) * inv, 1.0)
    s2p = jnp.where(nz, (2.0 * dx * dy) * inv, 0.0)
    cc = c2p * ca + s2p * sa
    ss = s2p * ca - c2p * sa
    cr = cb * ca + sb * sa
    sr = sb * ca - cb * sa
    st(rows_d, 16, dist)
    st(rows_d, 17, cc)
    st(rows_d, 18, ss)
    for c in range(VD):
        v = ld(rows_s, SD + c)
        st(rows_s, SD + c, v * cr)
        st(rows_s, SD + VD + c, v * sr)


def _gather_body(f_hbm, src_hbm, dst_hbm, gs_hbm, gd_hbm,
                 idx_s, idx_d, rows_s, rows_d, sem_g0, sem_g1, sem_w0,
                 sem_w1):
    sem_g = (sem_g0, sem_g1)
    sem_w = (sem_w0, sem_w1)
    wid = lax.axis_index("s") * NC + lax.axis_index("c")
    base = wid * EPW

    def fire_gathers(b, c):
        cps = []
        for j in range(G1_NB):
            isl = pl.ds(c * G1_CHUNK + j * G1_BATCH, G1_BATCH)
            osl = pl.ds(j * G1_BATCH, G1_BATCH)
            cps.append(pltpu.async_copy(
                f_hbm.at[idx_s.at[isl]], rows_s.at[b].at[osl], sem_g[b]))
            cps.append(pltpu.async_copy(
                f_hbm.at[idx_d.at[isl]], rows_d.at[b].at[osl], sem_g[b]))
        return cps

    def geo(b):
        def geo_body(k, carry2):
            for u in range(G1_UNROLL):
                _geo_group(rows_s.at[b], rows_d.at[b], k * G1_UNROLL + u)
            return carry2
        lax.fori_loop(0, G1_CHUNK // 16 // G1_UNROLL, geo_body, 0)

    def drain_writes(b):
        pltpu.make_async_copy(
            rows_s.at[b], gs_hbm.at[pl.ds(base, G1_CHUNK)], sem_w[b]).wait()
        pltpu.make_async_copy(
            rows_d.at[b], gd_hbm.at[pl.ds(base, G1_CHUNK)], sem_w[b]).wait()

    for b in range(2):
        pltpu.async_copy(rows_s.at[b], gs_hbm.at[pl.ds(base, G1_CHUNK)],
                         sem_w[b])
        pltpu.async_copy(rows_d.at[b], gd_hbm.at[pl.ds(base, G1_CHUNK)],
                         sem_w[b])

    def it_body(it, carry):
        soff = base + it * G1_SUP_E
        pltpu.sync_copy(src_hbm.at[pl.ds(soff, G1_SUP_E)], idx_s)
        pltpu.sync_copy(dst_hbm.at[pl.ds(soff, G1_SUP_E)], idx_d)
        drain_writes(0)
        cps = fire_gathers(0, 0)
        for c in range(G1_SUPER):
            b = c % 2
            for cp in cps:
                cp.wait()
            if c + 1 < G1_SUPER:
                drain_writes(1 - b)
                cps = fire_gathers(1 - b, c + 1)
            geo(b)
            off = soff + c * G1_CHUNK
            pltpu.async_copy(rows_s.at[b], gs_hbm.at[pl.ds(off, G1_CHUNK)],
                             sem_w[b])
            pltpu.async_copy(rows_d.at[b], gd_hbm.at[pl.ds(off, G1_CHUNK)],
                             sem_w[b])
        return carry

    lax.fori_loop(0, G1_ITERS, it_body, 0)
    drain_writes(0)
    drain_writes(1)


@functools.cache
def _gather_edges():
    return pl.kernel(
        _gather_body,
        out_type=(jax.ShapeDtypeStruct((N_EDGES, FW), jnp.float32),
                  jax.ShapeDtypeStruct((N_EDGES, FW), jnp.float32)),
        mesh=plsc.VectorSubcoreMesh(core_axis_name="c", subcore_axis_name="s",
                                    num_cores=NC, num_subcores=NS),
        scratch_types=[
            pltpu.VMEM((G1_SUP_E,), jnp.int32),
            pltpu.VMEM((G1_SUP_E,), jnp.int32),
            pltpu.VMEM((2, G1_CHUNK, FW), jnp.float32),
            pltpu.VMEM((2, G1_CHUNK, FW), jnp.float32),
            pltpu.SemaphoreType.DMA,
            pltpu.SemaphoreType.DMA,
            pltpu.SemaphoreType.DMA,
            pltpu.SemaphoreType.DMA,
        ],
        compiler_params=_SC_PARAMS,
    )


def _mlp_body(gs_ref, gd_ref, ws_ref, wd_ref, b1_ref, w2_ref, b2_ref,
              out_ref):
    dot = functools.partial(jnp.dot, preferred_element_type=jnp.float32)
    pre = (dot(gs_ref[...], ws_ref[...]) + dot(gd_ref[...], wd_ref[...])
           + b1_ref[...])
    h = pre * jax.nn.sigmoid(pre)
    out_ref[...] = dot(h, w2_ref[...]) + b2_ref[...]


def _run_mlp(gs, gd, ws, wd, b1, w2, b2):
    full = lambda shape: pl.BlockSpec(shape, lambda i: (0, 0))
    return pl.pallas_call(
        _mlp_body,
        grid=(MLP_GRID,),
        in_specs=[
            pl.BlockSpec((MLP_ROWS, 128), lambda i: (i, 0)),
            pl.BlockSpec((MLP_ROWS, 128), lambda i: (i, 0)),
            full((128, 128)), full((128, 128)), full((1, 128)),
            full((128, 128)), full((1, 128)),
        ],
        out_specs=pl.BlockSpec((MLP_ROWS, 128), lambda i: (i, 0)),
        out_shape=jax.ShapeDtypeStruct((N_EDGES // 4, 128), jnp.float32),
    )(gs, gd, ws, wd, b1, w2, b2)


def _scatter_body(msg_hbm, dst_hbm, init_hbm, out_hbm,
                  acc, msg_v, idx_lin, idx2, sem_m0, sem_m1):
    sem_m = (sem_m0, sem_m1)
    c = lax.axis_index("c")
    s = lax.axis_index("s")
    nbase = c * NPS
    tbase = s * EPT
    tlast = tbase + EPT - S3_CHUNK
    pltpu.sync_copy(init_hbm.at[c, pl.ds(s * RPT, RPT)],
                    acc.at[pl.ds(s * RPT, RPT)])
    plsc.subcore_barrier()

    def fire_msg(b, off):
        off = jnp.minimum(off, tlast)
        pltpu.async_copy(msg_hbm.at[pl.ds(off, S3_CHUNK)], msg_v.at[b],
                         sem_m[b])

    def drain_msg(b):
        pltpu.make_async_copy(
            msg_hbm.at[pl.ds(tbase, S3_CHUNK)], msg_v.at[b], sem_m[b]).wait()

    fire_msg(0, tbase)

    def it_body(it, carry):
        sbase = tbase + it * S3_SUP_E
        pltpu.sync_copy(dst_hbm.at[pl.ds(sbase, S3_SUP_E)], idx_lin)
        for ch in range(S3_SUPER):
            b = ch % 2
            drain_msg(b)
            fire_msg(1 - b, sbase + (ch + 1) * S3_CHUNK)
            for k in range(S3_RC):
                loc = idx_lin[pl.ds(ch * S3_CHUNK + k * 16, 16)] - nbase
                oob = (loc < 0) | (loc >= NPS)
                loc = jnp.where(oob, DUMMY_ROW, loc)
                o = k * 16
                idx2[o // S3_BATCH, pl.ds(o % S3_BATCH, 16)] = loc
            for bb in range(S3_NB):
                pltpu.sync_copy(
                    msg_v.at[b].at[pl.ds(bb * S3_BATCH, S3_BATCH)],
                    acc.at[idx2.at[bb]], add=True)
        return carry

    lax.fori_loop(0, S3_ITERS, it_body, 0)
    drain_msg(0)
    plsc.subcore_barrier()
    pltpu.sync_copy(acc.at[pl.ds(s * RPT, RPT)],
                    out_hbm.at[c, pl.ds(s * RPT, RPT)])


@functools.cache
def _scatter_edges():
    return pl.kernel(
        _scatter_body,
        out_type=jax.ShapeDtypeStruct((NC, NP_PAD, AW), jnp.float32),
        mesh=plsc.VectorSubcoreMesh(core_axis_name="c", subcore_axis_name="s",
                                    num_cores=NC, num_subcores=NS),
        scratch_types=[
            pltpu.VMEM_SHARED((NP_PAD, AW), jnp.float32),
            pltpu.VMEM((2, S3_CHUNK, AW), jnp.float32),
            pltpu.VMEM((S3_SUP_E,), jnp.int32),
            pltpu.VMEM((S3_NB, S3_BATCH), jnp.int32),
            pltpu.SemaphoreType.DMA,
            pltpu.SemaphoreType.DMA,
        ],
        compiler_params=_SC_PARAMS,
    )


def kernel(h_scalar, h_vector, edge_index, pos, orientation, W1, b1, W2, b2):
    wa = W1[:SD]
    wb = W1[SD:2 * SD]
    wv = W1[2 * SD:2 * SD + VD]
    wq = jnp.stack([wv[1::2], -wv[0::2]], axis=1).reshape(VD, MW)
    wg = W1[2 * SD + VD:]
    ws32 = jnp.concatenate(
        [jnp.concatenate([wa, wv, wq], axis=0),
         jnp.zeros((FW, FW - MW), jnp.float32)], axis=1)
    wd32 = jnp.concatenate(
        [jnp.concatenate([wb, wg, jnp.zeros((FW - SD - 3, MW), jnp.float32)],
                         axis=0),
         jnp.zeros((FW, FW - MW), jnp.float32)], axis=1)
    w232 = jnp.concatenate(
        [jnp.concatenate([W2, jnp.zeros((FW - MW, MW), jnp.float32)], axis=0),
         jnp.zeros((FW, FW - MW), jnp.float32)], axis=1)
    eye4 = jnp.eye(4, dtype=jnp.float32)
    ws_big = jnp.kron(eye4, ws32)
    wd_big = jnp.kron(eye4, wd32)
    w2_big = jnp.kron(eye4, w232)
    b1p = jnp.concatenate([b1, jnp.zeros((FW - MW,), jnp.float32)])
    b2p = jnp.concatenate([b2, jnp.zeros((FW - MW,), jnp.float32)])
    b1_big = jnp.tile(b1p, 4).reshape(1, 128)
    b2_big = jnp.tile(b2p, 4).reshape(1, 128)

    src = edge_index[0]
    dst = edge_index[1]
    f_table = _build_node_table(h_scalar, h_vector, pos, orientation)
    gs, gd = _gather_edges()(f_table, src, dst)
    gs = gs.reshape(N_EDGES // 4, 128)
    gd = gd.reshape(N_EDGES // 4, 128)
    msg = _run_mlp(gs, gd, ws_big, wd_big, b1_big, w2_big, b2_big)
    msg = msg.reshape(N_EDGES, AW)

    base = jnp.concatenate(
        [h_scalar, h_vector, jnp.zeros((N_NODES, AW - MW), jnp.float32)],
        axis=1).reshape(NC, NPS, AW)
    init = jnp.concatenate(
        [base, jnp.zeros((NC, NP_PAD - NPS, AW), jnp.float32)], axis=1)
    acc = _scatter_edges()(msg, dst, init)

    out_scalar = jnp.concatenate([acc[0, :NPS, :SD], acc[1, :NPS, :SD]], axis=0)
    out_vector = jnp.concatenate(
        [acc[0, :NPS, SD:MW], acc[1, :NPS, SD:MW]], axis=0)
    return (out_scalar, out_vector)
